# baseline (speedup 1.0000x reference)
import functools

import jax
import jax.numpy as jnp
import numpy as np
from jax import lax
from jax.experimental import pallas as pl
from jax.experimental.pallas import tpu as pltpu

NUM_ITEMS = 1000000
NUM_NEG = 200
NUM_POS = 50
BATCH = 4096

_MASK32 = 0xFFFFFFFF
_ROT_A = (13, 15, 26, 6)
_ROT_B = (17, 29, 16, 24)


def _threefry_key_schedule():
    def rotl(x, r):
        x = int(x) & _MASK32
        return ((x << r) | (x >> (32 - r))) & _MASK32

    def tf2x32(k0, k1, x0, x1):
        ks = (k0, k1, k0 ^ k1 ^ 0x1BD11BDA)
        x0 = (x0 + k0) & _MASK32
        x1 = (x1 + k1) & _MASK32
        for i in range(5):
            for r in (_ROT_A if i % 2 == 0 else _ROT_B):
                x0 = (x0 + x1) & _MASK32
                x1 = rotl(x1, r) ^ x0
            x0 = (x0 + ks[(i + 1) % 3]) & _MASK32
            x1 = (x1 + ks[(i + 2) % 3] + i + 1) & _MASK32
        return x0, x1

    f0, f1 = tf2x32(0, 0, 0, 12345)
    return tf2x32(f0, f1, 0, 1)


_K0, _K1 = _threefry_key_schedule()
_KS2 = _K0 ^ _K1 ^ 0x1BD11BDA
_INJ = (
    (_K1, _KS2 + 1),
    (_KS2, _K0 + 2),
    (_K0, _K1 + 3),
    (_K1, _KS2 + 4),
    (_KS2, _K0 + 5),
)

_NEG_LOG_PROB = float(-np.log(np.float32(NUM_ITEMS)))


def _i32(v):
    return jnp.int32(((v & _MASK32) ^ 0x80000000) - 0x80000000)


def _rotl(x, r):
    return lax.shift_left(x, _i32(r)) | lax.shift_right_logical(x, _i32(32 - r))


def _umod_1m_plus1(x):
    y = (lax.shift_right_logical(x, _i32(20)) * _i32(48576)) + (x & _i32(0xFFFFF))
    y = (lax.shift_right_logical(y, _i32(20)) * _i32(48576)) + (y & _i32(0xFFFFF))
    y = (lax.shift_right_logical(y, _i32(20)) * _i32(48576)) + (y & _i32(0xFFFFF))
    y = y + _i32(1)
    return jnp.where(y > _i32(1000000), y - _i32(1000000), y)


def _sampler_body(block_cols, pos_ref, items_ref, nprob_ref):
    shape = (NUM_NEG, block_cols)
    col0 = pl.program_id(0) * _i32(block_cols * NUM_NEG)
    x1 = (
        lax.broadcasted_iota(jnp.int32, shape, 1) * _i32(NUM_NEG)
        + lax.broadcasted_iota(jnp.int32, shape, 0)
        + (col0 + _i32(_K1))
    )

    x0 = jnp.full(shape, _i32(_K0), jnp.int32)
    for grp in range(5):
        for r in (_ROT_A if grp % 2 == 0 else _ROT_B):
            x0 = x0 + x1
            x1 = _rotl(x1, r) ^ x0
        inj0, inj1 = _INJ[grp]
        x0 = x0 + _i32(inj0)
        x1 = x1 + _i32(inj1)

    bits = x0 ^ x1
    items_ref[...] = _umod_1m_plus1(bits)
    nprob_ref[...] = jnp.full(shape, _NEG_LOG_PROB, jnp.float32)
    pos_ref[...] = jnp.full((NUM_POS, block_cols), _NEG_LOG_PROB, jnp.float32)


@jax.jit
def kernel(query, pos_items):
    del query, pos_items
    block_cols = 1024
    grid = (BATCH // block_cols,)
    pos_t, items_t, nprob_t = pl.pallas_call(
        functools.partial(_sampler_body, block_cols),
        grid=grid,
        compiler_params=pltpu.CompilerParams(
            dimension_semantics=("parallel",),
            skip_device_barrier=True,
        ),
        out_specs=[
            pl.BlockSpec((NUM_POS, block_cols), lambda i: (0, i)),
            pl.BlockSpec((NUM_NEG, block_cols), lambda i: (0, i)),
            pl.BlockSpec((NUM_NEG, block_cols), lambda i: (0, i)),
        ],
        out_shape=[
            jax.ShapeDtypeStruct((NUM_POS, BATCH), jnp.float32),
            jax.ShapeDtypeStruct((NUM_NEG, BATCH), jnp.int32),
            jax.ShapeDtypeStruct((NUM_NEG, BATCH), jnp.float32),
        ],
    )()
    return pos_t.T, items_t.T, nprob_t.T


# speedup vs baseline: 1.0176x; 1.0176x over previous
import functools

import jax
import jax.numpy as jnp
import numpy as np
from jax import lax
from jax.experimental import pallas as pl
from jax.experimental.pallas import tpu as pltpu

NUM_ITEMS = 1000000
NUM_NEG = 200
NUM_POS = 50
BATCH = 4096

_MASK32 = 0xFFFFFFFF
_ROT_A = (13, 15, 26, 6)
_ROT_B = (17, 29, 16, 24)


def _threefry_key_schedule():
    def rotl(x, r):
        x = int(x) & _MASK32
        return ((x << r) | (x >> (32 - r))) & _MASK32

    def tf2x32(k0, k1, x0, x1):
        ks = (k0, k1, k0 ^ k1 ^ 0x1BD11BDA)
        x0 = (x0 + k0) & _MASK32
        x1 = (x1 + k1) & _MASK32
        for i in range(5):
            for r in (_ROT_A if i % 2 == 0 else _ROT_B):
                x0 = (x0 + x1) & _MASK32
                x1 = rotl(x1, r) ^ x0
            x0 = (x0 + ks[(i + 1) % 3]) & _MASK32
            x1 = (x1 + ks[(i + 2) % 3] + i + 1) & _MASK32
        return x0, x1

    f0, f1 = tf2x32(0, 0, 0, 12345)
    return tf2x32(f0, f1, 0, 1)


_K0, _K1 = _threefry_key_schedule()
_KS2 = _K0 ^ _K1 ^ 0x1BD11BDA
_INJ = (
    (_K1, _KS2 + 1),
    (_KS2, _K0 + 2),
    (_K0, _K1 + 3),
    (_K1, _KS2 + 4),
    (_KS2, _K0 + 5),
)

_NEG_LOG_PROB = float(-np.log(np.float32(NUM_ITEMS)))


def _i32(v):
    return jnp.int32(((v & _MASK32) ^ 0x80000000) - 0x80000000)


def _rotl(x, r):
    return lax.shift_left(x, _i32(r)) | lax.shift_right_logical(x, _i32(32 - r))


def _umod_1m_plus1(x):
    y = (lax.shift_right_logical(x, _i32(20)) * _i32(48576)) + (x & _i32(0xFFFFF))
    y = (lax.shift_right_logical(y, _i32(20)) * _i32(48576)) + (y & _i32(0xFFFFF))
    y = (lax.shift_right_logical(y, _i32(20)) * _i32(48576)) + (y & _i32(0xFFFFF))
    y = y + _i32(1)
    return jnp.where(y > _i32(1000000), y - _i32(1000000), y)


def _sampler_body(block_cols, pos_ref, items_ref, nprob_ref):
    shape = (NUM_NEG, block_cols)
    col0 = pl.program_id(0) * _i32(block_cols * NUM_NEG)
    x1 = (
        lax.broadcasted_iota(jnp.int32, shape, 1) * _i32(NUM_NEG)
        + lax.broadcasted_iota(jnp.int32, shape, 0)
        + (col0 + _i32(_K1))
    )

    x0 = jnp.full(shape, _i32(_K0), jnp.int32)
    for grp in range(5):
        for r in (_ROT_A if grp % 2 == 0 else _ROT_B):
            x0 = x0 + x1
            x1 = _rotl(x1, r) ^ x0
        inj0, inj1 = _INJ[grp]
        x0 = x0 + _i32(inj0)
        x1 = x1 + _i32(inj1)

    bits = x0 ^ x1
    items_ref[...] = _umod_1m_plus1(bits)
    nprob_ref[...] = jnp.full(shape, _NEG_LOG_PROB, jnp.float32)
    pos_ref[...] = jnp.full((NUM_POS, block_cols), _NEG_LOG_PROB, jnp.float32)


@jax.jit
def kernel(query, pos_items):
    del query, pos_items
    block_cols = 512
    grid = (BATCH // block_cols,)
    pos_t, items_t, nprob_t = pl.pallas_call(
        functools.partial(_sampler_body, block_cols),
        grid=grid,
        compiler_params=pltpu.CompilerParams(
            dimension_semantics=("parallel",),
            skip_device_barrier=True,
        ),
        out_specs=[
            pl.BlockSpec((NUM_POS, block_cols), lambda i: (0, i)),
            pl.BlockSpec((NUM_NEG, block_cols), lambda i: (0, i)),
            pl.BlockSpec((NUM_NEG, block_cols), lambda i: (0, i)),
        ],
        out_shape=[
            jax.ShapeDtypeStruct((NUM_POS, BATCH), jnp.float32),
            jax.ShapeDtypeStruct((NUM_NEG, BATCH), jnp.int32),
            jax.ShapeDtypeStruct((NUM_NEG, BATCH), jnp.float32),
        ],
    )()
    return pos_t.T, items_t.T, nprob_t.T
